# baseline (speedup 1.0000x reference)
import jax
import jax.numpy as jnp
from jax.experimental import pallas as pl
from jax.experimental.pallas import tpu as pltpu

_B = 4096
_E = 256
_NC = 8192
_BLK = 256
_NBLK = _B // _BLK
_PBLK = 512
_NPBLK = _B // _PBLK
_NSTEP = _NPBLK + _NBLK


def _normalize_rows(x):
    n = jnp.sqrt(jnp.sum(x * x, axis=-1, keepdims=True))
    return x / jnp.maximum(n, 1e-12)


_LOG2E = 1.4426950408889634


def _side(z_blk, cb, y2, ones8, avg_acc):
    s = jax.lax.dot_general(z_blk, cb, (((1,), (1,)), ((), ())))
    x2 = jnp.sum(z_blk * z_blk, axis=1, keepdims=True)
    d2 = jnp.maximum(x2 + y2 - 2.0 * s, 1e-12)
    d2min = jnp.min(d2, axis=1, keepdims=True)
    idx = jnp.argmin(d2, axis=1)
    rec_sum = jnp.sum(d2min)
    d = d2 * jax.lax.rsqrt(d2)
    p = jnp.exp2(d * (-5.0 * _LOG2E))
    psum = jax.lax.dot_general(p, ones8, (((1,), (0,)), ((), ())))[:, :1]
    w = 1.0 / psum
    avg_acc[...] += jax.lax.dot_general(
        w.reshape(1, _BLK), p, (((1,), (0,)), ((), ())))
    return idx, rec_sum


def _fused_kernel(a_ref, b_ref, wa_ref, ba_ref, wb_ref, bb_ref, cb_ref,
                  out_ref, za_ref, zb_ref,
                  avg_a_acc, avg_b_acc, y2_ref, col_acc, sc_acc):
    i = pl.program_id(0)

    @pl.when(i < _NPBLK)
    def _proj():
        xa = jax.lax.dot_general(a_ref[...], wa_ref[...],
                                 (((1,), (1,)), ((), ())))
        za_ref[pl.ds(i * _PBLK, _PBLK), :] = _normalize_rows(xa + ba_ref[...])
        xb = jax.lax.dot_general(b_ref[...], wb_ref[...],
                                 (((1,), (1,)), ((), ())))
        zb_ref[pl.ds(i * _PBLK, _PBLK), :] = _normalize_rows(xb + bb_ref[...])

    @pl.when(i == 0)
    def _init():
        avg_a_acc[...] = jnp.zeros_like(avg_a_acc)
        avg_b_acc[...] = jnp.zeros_like(avg_b_acc)
        col_acc[...] = jnp.zeros_like(col_acc)
        cb0 = cb_ref[...]
        y2_ref[...] = jnp.sum(cb0 * cb0, axis=1)[None, :]
        for k in range(5):
            sc_acc[k] = 0.0

    @pl.when(i >= _NPBLK)
    def _main():
        j = i - _NPBLK
        cb = cb_ref[...]
        y2 = y2_ref[...]
        za_blk = za_ref[pl.ds(j * _BLK, _BLK), :]
        zb_blk = zb_ref[pl.ds(j * _BLK, _BLK), :]

        ones8 = jnp.full((_NC, 8), 1.0, dtype=jnp.float32)
        idx_a, rec_a = _side(za_blk, cb, y2, ones8, avg_a_acc)
        idx_b, rec_b = _side(zb_blk, cb, y2, ones8, avg_b_acc)
        match = jnp.sum((idx_a == idx_b).astype(jnp.float32))

        simu = jax.lax.dot_general(za_blk, zb_ref[...],
                                   (((1,), (1,)), ((), ())))
        mu = jnp.max(simu, axis=1, keepdims=True)
        rowexp = jnp.exp2((simu - mu) * (_LOG2E / 0.07))
        lse_row = mu[:, 0] / 0.07 + jnp.log(jnp.sum(rowexp, axis=1))
        col_acc[...] += jax.lax.dot_general(
            jnp.exp(mu.reshape(1, _BLK) / 0.07), rowexp,
            (((1,), (0,)), ((), ())))
        diag = jnp.sum(za_blk * zb_blk, axis=1) / 0.07

        sc_acc[0] += rec_a
        sc_acc[1] += rec_b
        sc_acc[2] += jnp.sum(lse_row - diag)
        sc_acc[3] += jnp.sum(diag)
        sc_acc[4] += match

    @pl.when(i == _NSTEP - 1)
    def _fini():
        avg_a = avg_a_acc[...] / _B
        avg_b = avg_b_acc[...] / _B
        ha = -jnp.sum(avg_a * jnp.log(avg_a + 1e-8))
        hb = -jnp.sum(avg_b * jnp.log(avg_b + 1e-8))
        rec = 1.25 * (sc_acc[0] + sc_acc[1]) / (_B * _E)
        lse_col_sum = jnp.sum(jnp.log(col_acc[...]))
        cm = (sc_acc[2] + (lse_col_sum - sc_acc[3])) / (2.0 * _B)
        div = (ha + hb) / 2.0
        loss = rec + 0.5 * cm - 0.1 * div
        mt = sc_acc[4] / _B
        lane = jax.lax.broadcasted_iota(jnp.int32, (1, 128), 1)
        out_ref[...] = jnp.where(lane == 0, loss,
                                 jnp.where(lane == 1, mt, 0.0))


def kernel(a, b, Wa, ba, Wb, bb, codebook):
    out = pl.pallas_call(
        _fused_kernel,
        grid=(_NSTEP,),
        in_specs=[
            pl.BlockSpec((_PBLK, a.shape[1]),
                         lambda i: (jnp.minimum(i, _NPBLK - 1), 0)),
            pl.BlockSpec((_PBLK, b.shape[1]),
                         lambda i: (jnp.minimum(i, _NPBLK - 1), 0)),
            pl.BlockSpec(Wa.shape, lambda i: (0, 0)),
            pl.BlockSpec((1, _E), lambda i: (0, 0)),
            pl.BlockSpec(Wb.shape, lambda i: (0, 0)),
            pl.BlockSpec((1, _E), lambda i: (0, 0)),
            pl.BlockSpec((_NC, _E), lambda i: (0, 0)),
        ],
        out_specs=pl.BlockSpec((1, 128), lambda i: (0, 0)),
        out_shape=jax.ShapeDtypeStruct((1, 128), jnp.float32),
        scratch_shapes=[
            pltpu.VMEM((_B, _E), jnp.float32),
            pltpu.VMEM((_B, _E), jnp.float32),
            pltpu.VMEM((1, _NC), jnp.float32),
            pltpu.VMEM((1, _NC), jnp.float32),
            pltpu.VMEM((1, _NC), jnp.float32),
            pltpu.VMEM((1, _B), jnp.float32),
            pltpu.SMEM((5,), jnp.float32),
        ],
    )(a, b, Wa, ba[None, :], Wb, bb[None, :], codebook)

    return out[0, 0], out[0, 1]


# speedup vs baseline: 1.1718x; 1.1718x over previous
import jax
import jax.numpy as jnp
from jax.experimental import pallas as pl
from jax.experimental.pallas import tpu as pltpu

_B = 4096
_E = 256
_NC = 8192
_BLK = 256
_NBLK = _B // _BLK
_PBLK = 512
_NPBLK = _B // _PBLK
_NSTEP = _NPBLK + _NBLK


def _normalize_rows(x):
    n = jnp.sqrt(jnp.sum(x * x, axis=-1, keepdims=True))
    return x / jnp.maximum(n, 1e-12)


_LOG2E = 1.4426950408889634


def _side(z_blk, cb, y2, avg_acc):
    s = jax.lax.dot_general(z_blk, cb, (((1,), (1,)), ((), ())))
    x2 = jnp.sum(z_blk * z_blk, axis=1, keepdims=True)
    d2 = jnp.maximum(x2 + y2 - 2.0 * s, 1e-12)
    d2min = jnp.min(d2, axis=1, keepdims=True)
    idx = jnp.argmin(d2, axis=1)
    rec_sum = jnp.sum(d2min)
    d = d2 * jax.lax.rsqrt(d2)
    p = jnp.exp2(d * (-5.0 * _LOG2E))
    w = 1.0 / jnp.sum(p, axis=1, keepdims=True)
    avg_acc[...] += jax.lax.dot_general(
        w.reshape(1, _BLK), p, (((1,), (0,)), ((), ())))
    return idx, rec_sum


def _fused_kernel(a_ref, b_ref, wa_ref, ba_ref, wb_ref, bb_ref, cb_ref,
                  out_ref, za_ref, zb_ref,
                  avg_a_acc, avg_b_acc, y2_ref, col_acc, sc_acc):
    i = pl.program_id(0)

    @pl.when(i < _NPBLK)
    def _proj():
        xa = jax.lax.dot_general(a_ref[...], wa_ref[...],
                                 (((1,), (1,)), ((), ())))
        za_ref[pl.ds(i * _PBLK, _PBLK), :] = _normalize_rows(xa + ba_ref[...])
        xb = jax.lax.dot_general(b_ref[...], wb_ref[...],
                                 (((1,), (1,)), ((), ())))
        zb_ref[pl.ds(i * _PBLK, _PBLK), :] = _normalize_rows(xb + bb_ref[...])

    @pl.when(i == 0)
    def _init():
        avg_a_acc[...] = jnp.zeros_like(avg_a_acc)
        avg_b_acc[...] = jnp.zeros_like(avg_b_acc)
        col_acc[...] = jnp.zeros_like(col_acc)
        cb0 = cb_ref[...]
        y2_ref[...] = jnp.sum(cb0 * cb0, axis=1)[None, :]
        for k in range(5):
            sc_acc[k] = 0.0

    @pl.when(i >= _NPBLK)
    def _main():
        j = i - _NPBLK
        cb = cb_ref[...]
        y2 = y2_ref[...]
        za_blk = za_ref[pl.ds(j * _BLK, _BLK), :]
        zb_blk = zb_ref[pl.ds(j * _BLK, _BLK), :]

        idx_a, rec_a = _side(za_blk, cb, y2, avg_a_acc)
        idx_b, rec_b = _side(zb_blk, cb, y2, avg_b_acc)
        match = jnp.sum((idx_a == idx_b).astype(jnp.float32))

        simu = jax.lax.dot_general(za_blk, zb_ref[...],
                                   (((1,), (1,)), ((), ())))
        mu = jnp.max(simu, axis=1, keepdims=True)
        rowexp = jnp.exp2((simu - mu) * (_LOG2E / 0.07))
        lse_row = mu[:, 0] / 0.07 + jnp.log(jnp.sum(rowexp, axis=1))
        col_acc[...] += jax.lax.dot_general(
            jnp.exp(mu.reshape(1, _BLK) / 0.07), rowexp,
            (((1,), (0,)), ((), ())))
        diag = jnp.sum(za_blk * zb_blk, axis=1) / 0.07

        sc_acc[0] += rec_a
        sc_acc[1] += rec_b
        sc_acc[2] += jnp.sum(lse_row - diag)
        sc_acc[3] += jnp.sum(diag)
        sc_acc[4] += match

    @pl.when(i == _NSTEP - 1)
    def _fini():
        avg_a = avg_a_acc[...] / _B
        avg_b = avg_b_acc[...] / _B
        ha = -jnp.sum(avg_a * jnp.log(avg_a + 1e-8))
        hb = -jnp.sum(avg_b * jnp.log(avg_b + 1e-8))
        rec = 1.25 * (sc_acc[0] + sc_acc[1]) / (_B * _E)
        lse_col_sum = jnp.sum(jnp.log(col_acc[...]))
        cm = (sc_acc[2] + (lse_col_sum - sc_acc[3])) / (2.0 * _B)
        div = (ha + hb) / 2.0
        loss = rec + 0.5 * cm - 0.1 * div
        mt = sc_acc[4] / _B
        lane = jax.lax.broadcasted_iota(jnp.int32, (1, 128), 1)
        out_ref[...] = jnp.where(lane == 0, loss,
                                 jnp.where(lane == 1, mt, 0.0))


def kernel(a, b, Wa, ba, Wb, bb, codebook):
    out = pl.pallas_call(
        _fused_kernel,
        grid=(_NSTEP,),
        in_specs=[
            pl.BlockSpec((_PBLK, a.shape[1]),
                         lambda i: (jnp.minimum(i, _NPBLK - 1), 0)),
            pl.BlockSpec((_PBLK, b.shape[1]),
                         lambda i: (jnp.minimum(i, _NPBLK - 1), 0)),
            pl.BlockSpec(Wa.shape, lambda i: (0, 0)),
            pl.BlockSpec((1, _E), lambda i: (0, 0)),
            pl.BlockSpec(Wb.shape, lambda i: (0, 0)),
            pl.BlockSpec((1, _E), lambda i: (0, 0)),
            pl.BlockSpec((_NC, _E), lambda i: (0, 0)),
        ],
        out_specs=pl.BlockSpec((1, 128), lambda i: (0, 0)),
        out_shape=jax.ShapeDtypeStruct((1, 128), jnp.float32),
        scratch_shapes=[
            pltpu.VMEM((_B, _E), jnp.float32),
            pltpu.VMEM((_B, _E), jnp.float32),
            pltpu.VMEM((1, _NC), jnp.float32),
            pltpu.VMEM((1, _NC), jnp.float32),
            pltpu.VMEM((1, _NC), jnp.float32),
            pltpu.VMEM((1, _B), jnp.float32),
            pltpu.SMEM((5,), jnp.float32),
        ],
    )(a, b, Wa, ba[None, :], Wb, bb[None, :], codebook)

    return out[0, 0], out[0, 1]


# speedup vs baseline: 1.2076x; 1.0306x over previous
import jax
import jax.numpy as jnp
from jax.experimental import pallas as pl
from jax.experimental.pallas import tpu as pltpu

_B = 4096
_E = 256
_NC = 8192
_BLK = 256
_NBLK = _B // _BLK
_PBLK = 512
_NPBLK = _B // _PBLK
_NSTEP = 3 * _NPBLK + _NBLK


def _normalize_rows(x):
    n = jnp.sqrt(jnp.sum(x * x, axis=-1, keepdims=True))
    return x / jnp.maximum(n, 1e-12)


_LOG2E = 1.4426950408889634


def _side(z_blk, cb, y2, avg_acc):
    s = jax.lax.dot_general(z_blk, cb, (((1,), (1,)), ((), ())))
    x2 = jnp.sum(z_blk * z_blk, axis=1, keepdims=True)
    d2 = jnp.maximum(x2 + y2 - 2.0 * s, 1e-12)
    d2min = jnp.min(d2, axis=1, keepdims=True)
    idx = jnp.argmin(d2, axis=1)
    rec_sum = jnp.sum(d2min)
    d = d2 * jax.lax.rsqrt(d2)
    p = jnp.exp2(d * (-5.0 * _LOG2E))
    w = 1.0 / jnp.sum(p, axis=1, keepdims=True)
    avg_acc[...] += jax.lax.dot_general(
        w.reshape(1, _BLK), p, (((1,), (0,)), ((), ())))
    return idx, rec_sum


def _fused_kernel(a_ref, b_ref, wa_ref, ba_ref, wb_ref, bb_ref, cb_ref,
                  out_ref, za_ref, zb_ref,
                  avg_a_acc, avg_b_acc, y2_ref, col_acc, sc_acc):
    i = pl.program_id(0)
    k = i // 3
    r = i % 3

    @pl.when((i < 3 * _NPBLK) & (r == 0))
    def _proj():
        xa = jax.lax.dot_general(a_ref[...], wa_ref[...],
                                 (((1,), (1,)), ((), ())))
        za_ref[pl.ds(k * _PBLK, _PBLK), :] = _normalize_rows(xa + ba_ref[...])
        xb = jax.lax.dot_general(b_ref[...], wb_ref[...],
                                 (((1,), (1,)), ((), ())))
        zb_ref[pl.ds(k * _PBLK, _PBLK), :] = _normalize_rows(xb + bb_ref[...])

    @pl.when(i == 0)
    def _init():
        avg_a_acc[...] = jnp.zeros_like(avg_a_acc)
        avg_b_acc[...] = jnp.zeros_like(avg_b_acc)
        col_acc[...] = jnp.zeros_like(col_acc)
        cb0 = cb_ref[...]
        y2_ref[...] = jnp.sum(cb0 * cb0, axis=1)[None, :]
        for t in range(5):
            sc_acc[t] = 0.0

    @pl.when((i < 3 * _NPBLK) & (r > 0))
    def _dist():
        j = 2 * k + (r - 1)
        cb = cb_ref[...]
        y2 = y2_ref[...]
        za_blk = za_ref[pl.ds(j * _BLK, _BLK), :]
        zb_blk = zb_ref[pl.ds(j * _BLK, _BLK), :]

        idx_a, rec_a = _side(za_blk, cb, y2, avg_a_acc)
        idx_b, rec_b = _side(zb_blk, cb, y2, avg_b_acc)
        match = jnp.sum((idx_a == idx_b).astype(jnp.float32))

        sc_acc[0] += rec_a
        sc_acc[1] += rec_b
        sc_acc[4] += match

    @pl.when(i >= 3 * _NPBLK)
    def _sim():
        j = i - 3 * _NPBLK
        za_blk = za_ref[pl.ds(j * _BLK, _BLK), :]
        zb_blk = zb_ref[pl.ds(j * _BLK, _BLK), :]

        simu = jax.lax.dot_general(za_blk, zb_ref[...],
                                   (((1,), (1,)), ((), ())))
        mu = jnp.max(simu, axis=1, keepdims=True)
        rowexp = jnp.exp2((simu - mu) * (_LOG2E / 0.07))
        lse_row = mu[:, 0] / 0.07 + jnp.log(jnp.sum(rowexp, axis=1))
        col_acc[...] += jax.lax.dot_general(
            jnp.exp(mu.reshape(1, _BLK) / 0.07), rowexp,
            (((1,), (0,)), ((), ())))
        diag = jnp.sum(za_blk * zb_blk, axis=1) / 0.07

        sc_acc[2] += jnp.sum(lse_row - diag)
        sc_acc[3] += jnp.sum(diag)

    @pl.when(i == _NSTEP - 1)
    def _fini():
        avg_a = avg_a_acc[...] / _B
        avg_b = avg_b_acc[...] / _B
        ha = -jnp.sum(avg_a * jnp.log(avg_a + 1e-8))
        hb = -jnp.sum(avg_b * jnp.log(avg_b + 1e-8))
        rec = 1.25 * (sc_acc[0] + sc_acc[1]) / (_B * _E)
        lse_col_sum = jnp.sum(jnp.log(col_acc[...]))
        cm = (sc_acc[2] + (lse_col_sum - sc_acc[3])) / (2.0 * _B)
        div = (ha + hb) / 2.0
        loss = rec + 0.5 * cm - 0.1 * div
        mt = sc_acc[4] / _B
        lane = jax.lax.broadcasted_iota(jnp.int32, (1, 128), 1)
        out_ref[...] = jnp.where(lane == 0, loss,
                                 jnp.where(lane == 1, mt, 0.0))


def kernel(a, b, Wa, ba, Wb, bb, codebook):
    out = pl.pallas_call(
        _fused_kernel,
        grid=(_NSTEP,),
        in_specs=[
            pl.BlockSpec((_PBLK, a.shape[1]),
                         lambda i: (jnp.minimum(i // 3, _NPBLK - 1), 0)),
            pl.BlockSpec((_PBLK, b.shape[1]),
                         lambda i: (jnp.minimum(i // 3, _NPBLK - 1), 0)),
            pl.BlockSpec(Wa.shape, lambda i: (0, 0)),
            pl.BlockSpec((1, _E), lambda i: (0, 0)),
            pl.BlockSpec(Wb.shape, lambda i: (0, 0)),
            pl.BlockSpec((1, _E), lambda i: (0, 0)),
            pl.BlockSpec((_NC, _E), lambda i: (0, 0)),
        ],
        out_specs=pl.BlockSpec((1, 128), lambda i: (0, 0)),
        out_shape=jax.ShapeDtypeStruct((1, 128), jnp.float32),
        scratch_shapes=[
            pltpu.VMEM((_B, _E), jnp.float32),
            pltpu.VMEM((_B, _E), jnp.float32),
            pltpu.VMEM((1, _NC), jnp.float32),
            pltpu.VMEM((1, _NC), jnp.float32),
            pltpu.VMEM((1, _NC), jnp.float32),
            pltpu.VMEM((1, _B), jnp.float32),
            pltpu.SMEM((5,), jnp.float32),
        ],
    )(a, b, Wa, ba[None, :], Wb, bb[None, :], codebook)

    return out[0, 0], out[0, 1]


# speedup vs baseline: 1.2599x; 1.0433x over previous
import jax
import jax.numpy as jnp
from jax.experimental import pallas as pl
from jax.experimental.pallas import tpu as pltpu

_B = 4096
_E = 256
_NC = 8192
_BLK = 256
_NBLK = _B // _BLK
_PBLK = 512
_NPBLK = _B // _PBLK
_NSTEP = 3 * _NPBLK + _NBLK


def _normalize_rows(x):
    n = jnp.sqrt(jnp.sum(x * x, axis=-1, keepdims=True))
    return x / jnp.maximum(n, 1e-12)


_LOG2E = 1.4426950408889634


def _side(z_blk, cb, y2, avg_acc):
    s = jax.lax.dot_general(z_blk, cb, (((1,), (1,)), ((), ())))
    x2 = jnp.sum(z_blk * z_blk, axis=1, keepdims=True)
    d2 = jnp.maximum(x2 + y2 - 2.0 * s, 1e-12)
    d2min = jnp.min(d2, axis=1, keepdims=True)
    idx = jnp.argmin(d2, axis=1)
    rec_sum = jnp.sum(d2min)
    d = d2 * jax.lax.rsqrt(d2)
    p = jnp.exp2(d * (-5.0 * _LOG2E))
    w = 1.0 / jnp.sum(p, axis=1, keepdims=True)
    avg_acc[...] += jax.lax.dot_general(
        w.reshape(1, _BLK), p, (((1,), (0,)), ((), ())))
    return idx, rec_sum


def _fused_kernel(a_ref, b_ref, wa_ref, ba_ref, wb_ref, bb_ref, cb_ref,
                  out_ref, za_ref, zb_ref,
                  avg_a_acc, avg_b_acc, y2_ref, col_acc, sc_acc):
    i = pl.program_id(0)
    k = i // 3
    r = i % 3

    @pl.when((i < 3 * _NPBLK) & (r == 0))
    def _proj():
        xa = jax.lax.dot_general(a_ref[...], wa_ref[...],
                                 (((1,), (1,)), ((), ())))
        za_ref[pl.ds(k * _PBLK, _PBLK), :] = _normalize_rows(xa + ba_ref[...])
        xb = jax.lax.dot_general(b_ref[...], wb_ref[...],
                                 (((1,), (1,)), ((), ())))
        zb_ref[pl.ds(k * _PBLK, _PBLK), :] = _normalize_rows(xb + bb_ref[...])

    @pl.when(i == 0)
    def _init():
        avg_a_acc[...] = jnp.zeros_like(avg_a_acc)
        avg_b_acc[...] = jnp.zeros_like(avg_b_acc)
        col_acc[...] = jnp.zeros_like(col_acc)
        cb0 = cb_ref[...]
        y2_ref[...] = jnp.sum(cb0 * cb0, axis=1)[None, :]
        for t in range(5):
            sc_acc[t] = 0.0

    @pl.when((i < 3 * _NPBLK) & (r > 0))
    def _dist():
        j = 2 * k + (r - 1)
        cb = cb_ref[...]
        y2 = y2_ref[...]
        za_blk = za_ref[pl.ds(j * _BLK, _BLK), :]
        zb_blk = zb_ref[pl.ds(j * _BLK, _BLK), :]

        idx_a, rec_a = _side(za_blk, cb, y2, avg_a_acc)
        idx_b, rec_b = _side(zb_blk, cb, y2, avg_b_acc)
        match = jnp.sum((idx_a == idx_b).astype(jnp.float32))

        sc_acc[0] += rec_a
        sc_acc[1] += rec_b
        sc_acc[4] += match

    @pl.when(i >= 3 * _NPBLK)
    def _sim():
        j = i - 3 * _NPBLK
        za_blk = za_ref[pl.ds(j * _BLK, _BLK), :]
        zb_blk = zb_ref[pl.ds(j * _BLK, _BLK), :]

        simu = jax.lax.dot_general(za_blk, zb_ref[...],
                                   (((1,), (1,)), ((), ())))
        rowexp = jnp.exp2(simu * (_LOG2E / 0.07))
        lse_row = jnp.log(jnp.sum(rowexp, axis=1))
        col_acc[...] += jax.lax.dot_general(
            jnp.full((1, _BLK), 1.0, dtype=jnp.float32), rowexp,
            (((1,), (0,)), ((), ())))
        diag = jnp.sum(za_blk * zb_blk, axis=1) / 0.07

        sc_acc[2] += jnp.sum(lse_row - diag)
        sc_acc[3] += jnp.sum(diag)

    @pl.when(i == _NSTEP - 1)
    def _fini():
        avg_a = avg_a_acc[...] / _B
        avg_b = avg_b_acc[...] / _B
        ha = -jnp.sum(avg_a * jnp.log(avg_a + 1e-8))
        hb = -jnp.sum(avg_b * jnp.log(avg_b + 1e-8))
        rec = 1.25 * (sc_acc[0] + sc_acc[1]) / (_B * _E)
        lse_col_sum = jnp.sum(jnp.log(col_acc[...]))
        cm = (sc_acc[2] + (lse_col_sum - sc_acc[3])) / (2.0 * _B)
        div = (ha + hb) / 2.0
        loss = rec + 0.5 * cm - 0.1 * div
        mt = sc_acc[4] / _B
        lane = jax.lax.broadcasted_iota(jnp.int32, (1, 128), 1)
        out_ref[...] = jnp.where(lane == 0, loss,
                                 jnp.where(lane == 1, mt, 0.0))


def kernel(a, b, Wa, ba, Wb, bb, codebook):
    out = pl.pallas_call(
        _fused_kernel,
        grid=(_NSTEP,),
        in_specs=[
            pl.BlockSpec((_PBLK, a.shape[1]),
                         lambda i: (jnp.minimum(i // 3, _NPBLK - 1), 0)),
            pl.BlockSpec((_PBLK, b.shape[1]),
                         lambda i: (jnp.minimum(i // 3, _NPBLK - 1), 0)),
            pl.BlockSpec(Wa.shape, lambda i: (0, 0)),
            pl.BlockSpec((1, _E), lambda i: (0, 0)),
            pl.BlockSpec(Wb.shape, lambda i: (0, 0)),
            pl.BlockSpec((1, _E), lambda i: (0, 0)),
            pl.BlockSpec((_NC, _E), lambda i: (0, 0)),
        ],
        out_specs=pl.BlockSpec((1, 128), lambda i: (0, 0)),
        out_shape=jax.ShapeDtypeStruct((1, 128), jnp.float32),
        scratch_shapes=[
            pltpu.VMEM((_B, _E), jnp.float32),
            pltpu.VMEM((_B, _E), jnp.float32),
            pltpu.VMEM((1, _NC), jnp.float32),
            pltpu.VMEM((1, _NC), jnp.float32),
            pltpu.VMEM((1, _NC), jnp.float32),
            pltpu.VMEM((1, _B), jnp.float32),
            pltpu.SMEM((5,), jnp.float32),
        ],
    )(a, b, Wa, ba[None, :], Wb, bb[None, :], codebook)

    return out[0, 0], out[0, 1]
